# baseline (speedup 1.0000x reference)
import functools

import jax
import jax.numpy as jnp
from jax import lax
from jax.experimental import pallas as pl
from jax.experimental.pallas import tpu as pltpu
from jax.experimental.pallas import tpu_sc as plsc

EMBED = 256
NH = 8
NP = 4
H_ = 100
W_ = 100
HW = H_ * W_
BS = 4
NQ = 900
HD = EMBED // NH
NW = 32
CH = 96
NTERM = NP * 4
SC_CHUNKS = [(k * CH, CH, CH) for k in range(NQ // CH)] + [(864, 40, 36)]

POS_TILE = 1000


def _value_proj_body(v_ref, wt_ref, b_ref, out_ref):
    acc = jnp.dot(v_ref[0], wt_ref[...], preferred_element_type=jnp.float32)
    acc = acc + b_ref[...][None, :]
    out_ref[0] = acc[:, 0:128]
    out_ref[1] = acc[:, 128:256]


def _value_proj(value, w_val_t, b_val):
    nt = HW // POS_TILE
    return pl.pallas_call(
        _value_proj_body,
        grid=(BS, nt),
        in_specs=[
            pl.BlockSpec((1, POS_TILE, EMBED), lambda b, t: (b, t, 0)),
            pl.BlockSpec((EMBED, EMBED), lambda b, t: (0, 0)),
            pl.BlockSpec((EMBED,), lambda b, t: (0,)),
        ],
        out_specs=pl.BlockSpec((2, POS_TILE, 128),
                               lambda b, t: (0, b * nt + t, 0)),
        out_shape=jax.ShapeDtypeStruct((2, BS * HW, 128), jnp.float32),
    )(value, w_val_t, b_val)


def _query_side_body(q_ref, r_ref, wx_ref, bx_ref, wy_ref, by_ref,
                     wa_ref, ba_ref, g_ref, p_ref, idx_ref, wt_ref):
    b = pl.program_id(0)
    q = q_ref[0]
    refx = r_ref[0, :, 0:1]
    refy = r_ref[0, :, 1:2]

    px = jnp.dot(q, wx_ref[...].T, preferred_element_type=jnp.float32)
    px = px + bx_ref[...][None, :] + (refx * W_ - 0.5)
    py = jnp.dot(q, wy_ref[...].T, preferred_element_type=jnp.float32)
    py = py + by_ref[...][None, :] + (refy * H_ - 0.5)

    x0 = jnp.clip(jnp.floor(px), 0.0, W_ - 2.0)
    y0 = jnp.clip(jnp.floor(py), 0.0, H_ - 2.0)

    head32 = lax.broadcasted_iota(jnp.int32, (NQ, NH * NP), 1) // NP
    x0i = x0.astype(jnp.int32)
    y0i = y0.astype(jnp.int32)

    logits = jnp.dot(q, wa_ref[...].T, preferred_element_type=jnp.float32)
    logits = logits + ba_ref[...][None, :]
    m = jnp.max(logits, axis=1, keepdims=True)
    e = jnp.exp(logits - m)
    s = jnp.dot(e, g_ref[...], preferred_element_type=jnp.float32)
    aw = e / s

    x128 = jnp.zeros((NQ, NH * NTERM), jnp.float32)
    y128 = jnp.zeros((NQ, NH * NTERM), jnp.float32)
    w128 = jnp.zeros((NQ, NH * NTERM), jnp.float32)
    for sub in range(4):
        sx = sub % 2
        sy = sub // 2
        tx = jnp.maximum(0.0, 1.0 - jnp.abs(px - (x0 + float(sx))))
        ty = jnp.maximum(0.0, 1.0 - jnp.abs(py - (y0 + float(sy))))
        perm = p_ref[sub]
        w128 = w128 + jnp.dot(aw * tx * ty, perm,
                              preferred_element_type=jnp.float32)
        x128 = x128 + jnp.dot(x0 + float(sx), perm,
                              preferred_element_type=jnp.float32)
        y128 = y128 + jnp.dot(y0 + float(sy), perm,
                              preferred_element_type=jnp.float32)
    wt_ref[0] = w128
    head128 = lax.broadcasted_iota(jnp.int32, (NQ, NH * NTERM), 1) // NTERM
    pos = y128.astype(jnp.int32) * W_ + x128.astype(jnp.int32)
    idx_ref[0] = ((head128 // 4) * (BS * HW * 4)
                  + (b * HW + pos) * 4 + head128 % 4)


def _query_side(query, ref_pts, wx, bx, wy, by, wa, ba, g, perms):
    n128 = NH * NP * 4
    return pl.pallas_call(
        _query_side_body,
        grid=(BS,),
        in_specs=[
            pl.BlockSpec((1, NQ, EMBED), lambda b: (b, 0, 0)),
            pl.BlockSpec((1, NQ, 2), lambda b: (b, 0, 0)),
            pl.BlockSpec((NH * NP, EMBED), lambda b: (0, 0)),
            pl.BlockSpec((NH * NP,), lambda b: (0,)),
            pl.BlockSpec((NH * NP, EMBED), lambda b: (0, 0)),
            pl.BlockSpec((NH * NP,), lambda b: (0,)),
            pl.BlockSpec((NH * NP, EMBED), lambda b: (0, 0)),
            pl.BlockSpec((NH * NP,), lambda b: (0,)),
            pl.BlockSpec((NH * NP, NH * NP), lambda b: (0, 0)),
            pl.BlockSpec((4, NH * NP, n128), lambda b: (0, 0, 0)),
        ],
        out_specs=[
            pl.BlockSpec((1, NQ, n128), lambda b: (b, 0, 0)),
            pl.BlockSpec((1, NQ, n128), lambda b: (b, 0, 0)),
        ],
        out_shape=[
            jax.ShapeDtypeStruct((BS, NQ, n128), jnp.int32),
            jax.ShapeDtypeStruct((BS, NQ, n128), jnp.float32),
        ],
    )(query, ref_pts, wx, bx, wy, by, wa, ba, g, perms)


def _sc_gather_reduce(vt_flat, idxr, wtr):
    mesh = plsc.VectorSubcoreMesh(core_axis_name="c", subcore_axis_name="s")

    @functools.partial(
        pl.kernel,
        mesh=mesh,
        compiler_params=pltpu.CompilerParams(use_tc_tiling_on_sc=False),
        out_type=jax.ShapeDtypeStruct((2, BS * NQ, 128), jnp.float32),
        scratch_types=[
            pltpu.VMEM((2, CH, NTERM), jnp.int32),
            pltpu.VMEM((2, CH * NTERM), jnp.int32),
            pltpu.VMEM((2, CH * NTERM, HD), jnp.float32),
            pltpu.VMEM((2, CH, NTERM), jnp.float32),
            pltpu.VMEM((CH, HD), jnp.float32),
            pltpu.SemaphoreType.DMA,
            pltpu.SemaphoreType.DMA,
        ],
    )
    def body(vt_hbm, idx_hbm, wt_hbm, out_hbm,
             idx_v, flat_v, rows_v, wt_v, out_v, sem0, sem1):
        w = lax.axis_index("s") * 2 + lax.axis_index("c")
        b = w // NH
        h = w % NH
        sems = [sem0, sem1]

        def fetch(q0, gl, ib):
            pltpu.sync_copy(
                idx_hbm.at[b, pl.ds(q0, gl), pl.ds(h * NTERM, NTERM)],
                idx_v.at[ib, pl.ds(0, gl)])
            pltpu.sync_copy(
                wt_hbm.at[b, pl.ds(q0, gl), pl.ds(h * NTERM, NTERM)],
                wt_v.at[ib, pl.ds(0, gl)])

            def repack(qi, c2):
                flat_v[ib, pl.ds(qi * NTERM, NTERM)] = idx_v[ib, qi, :]
                return c2

            lax.fori_loop(0, gl, repack, 0)
            handles = []
            for j in range(gl * NTERM // 128):
                handles.append(pltpu.async_copy(
                    vt_hbm.at[flat_v.at[ib, pl.ds(j * 128, 128)]],
                    rows_v.at[ib, pl.ds(j * 128, 128)], sems[ib]))
            return handles

        def compute(q0, ql, ib):
            def qstep(qi, c2):
                wrow = wt_v[ib, qi, :]
                acc0 = jnp.zeros((16,), jnp.float32)
                acc1 = jnp.zeros((16,), jnp.float32)
                for t in range(NTERM):
                    wsc = wrow[t]
                    acc0 = acc0 + wsc * rows_v[ib, qi * NTERM + t,
                                               pl.ds(0, 16)]
                    acc1 = acc1 + wsc * rows_v[ib, qi * NTERM + t,
                                               pl.ds(16, 16)]
                out_v[qi, pl.ds(0, 16)] = acc0
                out_v[qi, pl.ds(16, 16)] = acc1
                return c2

            lax.fori_loop(0, ql, qstep, 0)
            pltpu.sync_copy(
                out_v.at[pl.ds(0, ql)],
                out_hbm.at[h // 4, pl.ds(b * NQ + q0, ql),
                           pl.ds((h % 4) * HD, HD)])

        handles = fetch(SC_CHUNKS[0][0], SC_CHUNKS[0][1], 0)
        for i, (q0, gl, ql) in enumerate(SC_CHUNKS):
            for hnd in handles:
                hnd.wait()
            if i + 1 < len(SC_CHUNKS):
                nq0, ngl, _ = SC_CHUNKS[i + 1]
                handles = fetch(nq0, ngl, (i + 1) % 2)
            compute(q0, ql, i % 2)

    return body(vt_flat, idxr, wtr)


CQ = 1200


def _out_proj_body(s_ref, wt_ref, b_ref, o_ref):
    s = jnp.concatenate([s_ref[0], s_ref[1]], axis=1)
    acc = jnp.dot(s, wt_ref[...], preferred_element_type=jnp.float32)
    o_ref[...] = acc + b_ref[...][None, :]


def _out_proj(sampled2, w_out_t, b_out):
    return pl.pallas_call(
        _out_proj_body,
        grid=(BS * NQ // CQ,),
        in_specs=[
            pl.BlockSpec((2, CQ, 128), lambda i: (0, i, 0)),
            pl.BlockSpec((EMBED, EMBED), lambda i: (0, 0)),
            pl.BlockSpec((EMBED,), lambda i: (0,)),
        ],
        out_specs=pl.BlockSpec((CQ, EMBED), lambda i: (i, 0)),
        out_shape=jax.ShapeDtypeStruct((BS * NQ, EMBED), jnp.float32),
    )(sampled2, w_out_t, b_out)


def kernel(query, value, reference_points, spatial_shapes,
           W_off, b_off, W_attn, b_attn, W_val, b_val, W_out, b_out):
    f32 = jnp.float32
    j32 = jnp.arange(NH * NP)
    wx = W_off[j32 * 2].astype(f32)
    bx = b_off[j32 * 2].astype(f32)
    wy = W_off[j32 * 2 + 1].astype(f32)
    by = b_off[j32 * 2 + 1].astype(f32)
    wa = W_attn.astype(f32)
    ba = b_attn.astype(f32)
    head32a = j32 // NP
    g = (head32a[:, None] == head32a[None, :]).astype(f32)
    s4 = jnp.arange(4)[:, None, None]
    j128b = jnp.arange(NH * NTERM)[None, None, :]
    c32 = j32[None, :, None]
    perms = (j128b == (c32 // NP) * NTERM + s4 * NP + c32 % NP).astype(f32)

    vproj = _value_proj(value, W_val.T.astype(f32), b_val.astype(f32))
    vt_flat = vproj.reshape(BS * HW * NH, HD)

    ref_pts = reference_points[:, :, 0, :]
    idx, wt = _query_side(query, ref_pts, wx, bx, wy, by, wa, ba, g, perms)

    idxr = jnp.pad(idx, ((0, 0), (0, 4), (0, 0)))
    wtr = jnp.pad(wt, ((0, 0), (0, 4), (0, 0)))

    sampled = _sc_gather_reduce(vt_flat, idxr, wtr)

    out2d = _out_proj(sampled, W_out.T.astype(f32), b_out.astype(f32))
    return out2d.reshape(BS, NQ, EMBED)
